# baseline (speedup 1.0000x reference)
import functools

import jax
import jax.numpy as jnp
from jax import lax
from jax.experimental import pallas as pl
from jax.experimental.pallas import tpu as pltpu
from jax.experimental.pallas import tpu_sc as plsc

B = 16384
D = 64
NUM_PTM = 26
PER = 128
K = NUM_PTM * PER
COMMIT = 0.25
TEMP = 0.07
SAMPLED = int(0.1 * PER) * NUM_PTM
SPAD = 384

DPAD = 128
TBP = 1024
TBS = 512
TBL = 4096
NT = 32
N_SC_CORES = 2
N_SC_SUBCORES = 16
NW = N_SC_CORES * N_SC_SUBCORES
BPW = B // NW


def _perm_body(q_ref, lt_ref, slot_ref, starts_ref, hist_ref, carry_ref):
    i = pl.program_id(0)
    nbp = B // TBP
    t_iota = lax.broadcasted_iota(jnp.int32, (1, NT), 1)
    oneh = (q_ref[...] == t_iota).astype(jnp.float32)

    @pl.when(i == 0)
    def _():
        hist_ref[...] = jnp.zeros_like(hist_ref)

    @pl.when(i < nbp)
    def _():
        hist_ref[...] += jnp.sum(oneh, axis=0, keepdims=True)

    @pl.when(i == nbp)
    def _():
        h = hist_ref[...]
        strict_lt = (lax.broadcasted_iota(jnp.int32, (NT, NT), 0)
                     < lax.broadcasted_iota(jnp.int32, (NT, NT), 1))
        st = lax.dot_general(h, strict_lt.astype(jnp.float32),
                             (((1,), (0,)), ((), ())),
                             preferred_element_type=jnp.float32,
                             precision=lax.Precision.HIGHEST)
        starts_ref[...] = st
        carry_ref[...] = st

    @pl.when(i >= nbp)
    def _():
        pref = lax.dot_general(lt_ref[...], oneh, (((1,), (0,)), ((), ())),
                               preferred_element_type=jnp.float32,
                               precision=lax.Precision.HIGHEST)
        base = carry_ref[...]
        pos = jnp.sum(oneh * (base + pref), axis=1, keepdims=True) - 1.0
        slot_ref[...] = pos.astype(jnp.int32)
        carry_ref[...] = base + jnp.sum(oneh, axis=0, keepdims=True)


def _tc_perm(q2, lt, interpret=False):
    nbp = B // TBP
    return pl.pallas_call(
        _perm_body,
        grid=(2 * nbp,),
        in_specs=[
            pl.BlockSpec((TBP, 1), lambda i: (i % (B // TBP), 0)),
            pl.BlockSpec((TBP, TBP), lambda i: (0, 0)),
        ],
        out_specs=[
            pl.BlockSpec((TBP, 1), lambda i: (i % (B // TBP), 0)),
            pl.BlockSpec((1, NT), lambda i: (0, 0)),
        ],
        out_shape=[
            jax.ShapeDtypeStruct((B, 1), jnp.int32),
            jax.ShapeDtypeStruct((1, NT), jnp.float32),
        ],
        scratch_shapes=[
            pltpu.VMEM((1, NT), jnp.float32),
            pltpu.VMEM((1, NT), jnp.float32),
        ],
        interpret=interpret,
    )(q2, lt)


def _sc_scatter_x(xpad, slotof):
    mesh = plsc.VectorSubcoreMesh(core_axis_name="c", subcore_axis_name="s")

    @functools.partial(
        pl.kernel,
        out_type=jax.ShapeDtypeStruct((B, DPAD), jnp.float32),
        mesh=mesh,
        scratch_types=[
            pltpu.VMEM((BPW,), jnp.int32),
            pltpu.VMEM((BPW, DPAD), jnp.float32),
            pltpu.SemaphoreType.DMA,
        ],
    )
    def sk(xpad_hbm, slotof_hbm, xs_hbm, ids_v, rows_v, sem):
        wid = lax.axis_index("s") * N_SC_CORES + lax.axis_index("c")
        base = wid * BPW
        pltpu.sync_copy(slotof_hbm.at[pl.ds(base, BPW)], ids_v)
        pltpu.sync_copy(xpad_hbm.at[pl.ds(base, BPW)], rows_v)
        pltpu.async_copy(rows_v, xs_hbm.at[ids_v], sem).wait()

    return sk(xpad, slotof)


def _bucket_body(xs_ref, starts_ref, tab_ref, e2_ref, sidx_ref,
                 enc_ref, tablen_ref, ul_ref):
    step = pl.program_id(0)

    xr = xs_ref[:, :D]
    norm = jnp.sqrt(jnp.sum(xr * xr, axis=1, keepdims=True))
    xn = xr / jnp.maximum(norm, 1e-12)
    c = jnp.sum(xn * xn, axis=1, keepdims=True)

    fslot = (lax.broadcasted_iota(jnp.int32, (TBS, 1), 0)
             + step * TBS).astype(jnp.float32)
    qs = (jnp.sum((fslot >= starts_ref[...]).astype(jnp.int32),
                  axis=1, keepdims=True) - 1)
    t_lo = jnp.min(qs)
    t_hi = jnp.max(qs)

    def tb(t, best):
        st = tab_ref[pl.ds(t * PER, PER), :]
        sc = lax.dot_general(xn, st, (((1,), (1,)), ((), ())),
                             preferred_element_type=jnp.float32)
        e2r = e2_ref[pl.ds(t * 8, 8), :][0:1, :]
        d2b = (c + e2r) - 2.0 * sc
        mb = jnp.min(d2b, axis=1, keepdims=True)
        lane = lax.broadcasted_iota(jnp.int32, (TBS, PER), 1)
        ib = jnp.min(jnp.where(d2b == mb, lane, PER), axis=1, keepdims=True)
        return jnp.where(qs == t, t * PER + ib, best)

    best = lax.fori_loop(t_lo, t_hi + 1, tb, jnp.zeros((TBS, 1), jnp.int32))
    enc_ref[...] = best

    @pl.when(step == 0)
    def _():
        tab = tab_ref[...]
        tn = tab / jnp.maximum(
            jnp.sqrt(jnp.sum(tab * tab, axis=1, keepdims=True)), 1e-12)
        tablen_ref[...] = jnp.concatenate(
            [tn, jnp.zeros((K, DPAD - D), jnp.float32)], axis=1)
        si = sidx_ref[...]
        onehot = (si.reshape(SPAD, 1)
                  == lax.broadcasted_iota(jnp.int32, (SPAD, K), 1))
        se = lax.dot_general(onehot.astype(jnp.float32), tn,
                             (((1,), (0,)), ((), ())),
                             preferred_element_type=jnp.float32,
                             precision=lax.Precision.HIGHEST)
        sim = lax.dot_general(se, se, (((1,), (1,)), ((), ())),
                              preferred_element_type=jnp.float32,
                              precision=lax.Precision.HIGHEST)
        valid = si.reshape(1, SPAD) >= 0
        eye = (lax.broadcasted_iota(jnp.int32, (SPAD, SPAD), 0)
               == lax.broadcasted_iota(jnp.int32, (SPAD, SPAD), 1))
        keep = valid & jnp.logical_not(eye)
        simm = jnp.where(keep, sim, -jnp.float32(jnp.inf))
        ex = jnp.exp(simm / TEMP)
        sum_exp = jnp.sum(ex, axis=1, keepdims=True)
        labels = jnp.where(si >= 0, si // PER, -1)
        pos = labels.reshape(SPAD, 1) == labels.reshape(1, SPAD)
        pos_sum = jnp.sum(jnp.where(pos, ex, 0.0), axis=1, keepdims=True)
        validc = si.reshape(SPAD, 1) >= 0
        ratio = jnp.where(validc, pos_sum / jnp.maximum(sum_exp, 1e-30), 1.0)
        ul = -jnp.sum(jnp.log(ratio)) / SAMPLED
        ul_ref[...] = jnp.full((1, 1), ul, dtype=jnp.float32)


def _tc_bucket(xs, starts, table, e2rep, sidx, interpret=False):
    return pl.pallas_call(
        _bucket_body,
        grid=(B // TBS,),
        in_specs=[
            pl.BlockSpec((TBS, DPAD), lambda i: (i, 0)),
            pl.BlockSpec((1, NT), lambda i: (0, 0)),
            pl.BlockSpec((K, D), lambda i: (0, 0)),
            pl.BlockSpec((NUM_PTM * 8, PER), lambda i: (0, 0)),
            pl.BlockSpec((1, SPAD), lambda i: (0, 0)),
        ],
        out_specs=[
            pl.BlockSpec((TBS, 1), lambda i: (i, 0)),
            pl.BlockSpec((K, DPAD), lambda i: (0, 0)),
            pl.BlockSpec((1, 1), lambda i: (0, 0)),
        ],
        out_shape=[
            jax.ShapeDtypeStruct((B, 1), jnp.int32),
            jax.ShapeDtypeStruct((K, DPAD), jnp.float32),
            jax.ShapeDtypeStruct((1, 1), jnp.float32),
        ],
        interpret=interpret,
    )(xs, starts, table, e2rep, sidx)


def _sc_gather_out(tablen, encs, slotof):
    mesh = plsc.VectorSubcoreMesh(core_axis_name="c", subcore_axis_name="s")

    @functools.partial(
        pl.kernel,
        out_type=[jax.ShapeDtypeStruct((B,), jnp.int32),
                  jax.ShapeDtypeStruct((B, DPAD), jnp.float32)],
        mesh=mesh,
        scratch_types=[
            pltpu.VMEM((BPW,), jnp.int32),
            pltpu.VMEM((BPW,), jnp.int32),
            pltpu.VMEM((BPW, DPAD), jnp.float32),
            pltpu.SemaphoreType.DMA,
            pltpu.SemaphoreType.DMA,
        ],
    )
    def gk(tablen_hbm, encs_hbm, slotof_hbm, enc_hbm, quant_hbm,
           ids_v, enc_v, rows_v, sem1, sem2):
        wid = lax.axis_index("s") * N_SC_CORES + lax.axis_index("c")
        base = wid * BPW
        pltpu.sync_copy(slotof_hbm.at[pl.ds(base, BPW)], ids_v)
        pltpu.async_copy(encs_hbm.at[ids_v], enc_v, sem1).wait()
        pltpu.sync_copy(enc_v, enc_hbm.at[pl.ds(base, BPW)])
        pltpu.async_copy(tablen_hbm.at[enc_v], rows_v, sem2).wait()
        pltpu.sync_copy(rows_v, quant_hbm.at[pl.ds(base, BPW)])

    return gk(tablen, encs, slotof)


def _loss_body(x_ref, qp_ref, lsum_ref):
    step = pl.program_id(0)
    xr = x_ref[...]
    norm = jnp.sqrt(jnp.sum(xr * xr, axis=1, keepdims=True))
    xn = xr / jnp.maximum(norm, 1e-12)
    diff = qp_ref[:, :D] - xn

    @pl.when(step == 0)
    def _():
        lsum_ref[...] = jnp.zeros_like(lsum_ref)

    lsum_ref[...] += jnp.sum(diff * diff).reshape(1, 1)


def _tc_loss(x, qp, interpret=False):
    return pl.pallas_call(
        _loss_body,
        grid=(B // TBL,),
        in_specs=[
            pl.BlockSpec((TBL, D), lambda i: (i, 0)),
            pl.BlockSpec((TBL, DPAD), lambda i: (i, 0)),
        ],
        out_specs=pl.BlockSpec((1, 1), lambda i: (0, 0)),
        out_shape=jax.ShapeDtypeStruct((1, 1), jnp.float32),
        interpret=interpret,
    )(x, qp)


def _sampled_indices():
    perm = jax.random.permutation(jax.random.key(42), PER)[:int(0.1 * PER)]
    all_idx = jnp.arange(K).reshape(NUM_PTM, PER)
    si = all_idx[:, perm].reshape(-1).astype(jnp.int32)
    return jnp.concatenate(
        [si, jnp.full((SPAD - SAMPLED,), -1, jnp.int32)]).reshape(1, SPAD)


def kernel(x, Q, embeddings):
    q2 = Q.reshape(B, 1)
    xpad = jnp.concatenate([x, jnp.zeros((B, DPAD - D), x.dtype)], axis=1)
    e2rep = jnp.repeat(jnp.sum(embeddings ** 2, axis=1).reshape(NUM_PTM, PER),
                       8, axis=0)
    sidx = _sampled_indices()
    lt = jnp.tril(jnp.ones((TBP, TBP), jnp.float32))

    slotof2, starts = _tc_perm(q2, lt)
    slotof = slotof2.reshape(B)
    xs = _sc_scatter_x(xpad, slotof)
    encs2, tablen, ul = _tc_bucket(xs, starts, embeddings, e2rep, sidx)
    enc, qp = _sc_gather_out(tablen, encs2.reshape(B), slotof)
    lsum = _tc_loss(x, qp)

    quantized = qp[:, :D]
    loss = lsum[0, 0] * ((1.0 + COMMIT) / (B * D))
    return (quantized, loss, ul[0, 0], enc)


# speedup vs baseline: 1.0381x; 1.0381x over previous
import functools

import jax
import jax.numpy as jnp
from jax import lax
from jax.experimental import pallas as pl
from jax.experimental.pallas import tpu as pltpu
from jax.experimental.pallas import tpu_sc as plsc

B = 16384
D = 64
NUM_PTM = 26
PER = 128
K = NUM_PTM * PER
COMMIT = 0.25
TEMP = 0.07
SAMPLED = int(0.1 * PER) * NUM_PTM
SPAD = 384

DPAD = 128
TBP = 512
TBS = 1024
TBL = 4096
NT = 32
N_SC_CORES = 2
N_SC_SUBCORES = 16
NW = N_SC_CORES * N_SC_SUBCORES
BPW = B // NW


def _perm_body(q_ref, lt_ref, slot_ref, starts_ref, hist_ref, carry_ref):
    i = pl.program_id(0)
    nbp = B // TBP
    t_iota = lax.broadcasted_iota(jnp.int32, (1, NT), 1)
    oneh = (q_ref[...] == t_iota).astype(jnp.float32)

    @pl.when(i == 0)
    def _():
        hist_ref[...] = jnp.zeros_like(hist_ref)

    @pl.when(i < nbp)
    def _():
        hist_ref[...] += jnp.sum(oneh, axis=0, keepdims=True)

    @pl.when(i == nbp)
    def _():
        h = hist_ref[...]
        strict_lt = (lax.broadcasted_iota(jnp.int32, (NT, NT), 0)
                     < lax.broadcasted_iota(jnp.int32, (NT, NT), 1))
        st = lax.dot_general(h, strict_lt.astype(jnp.float32),
                             (((1,), (0,)), ((), ())),
                             preferred_element_type=jnp.float32,
                             precision=lax.Precision.HIGHEST)
        starts_ref[...] = st
        carry_ref[...] = st

    @pl.when(i >= nbp)
    def _():
        pref = lax.dot_general(lt_ref[...], oneh, (((1,), (0,)), ((), ())),
                               preferred_element_type=jnp.float32,
                               precision=lax.Precision.HIGHEST)
        base = carry_ref[...]
        pos = jnp.sum(oneh * (base + pref), axis=1, keepdims=True) - 1.0
        slot_ref[...] = pos.astype(jnp.int32)
        carry_ref[...] = base + jnp.sum(oneh, axis=0, keepdims=True)


def _tc_perm(q2, lt, interpret=False):
    nbp = B // TBP
    return pl.pallas_call(
        _perm_body,
        grid=(2 * nbp,),
        in_specs=[
            pl.BlockSpec((TBP, 1), lambda i: (i % (B // TBP), 0)),
            pl.BlockSpec((TBP, TBP), lambda i: (0, 0)),
        ],
        out_specs=[
            pl.BlockSpec((TBP, 1), lambda i: (i % (B // TBP), 0)),
            pl.BlockSpec((1, NT), lambda i: (0, 0)),
        ],
        out_shape=[
            jax.ShapeDtypeStruct((B, 1), jnp.int32),
            jax.ShapeDtypeStruct((1, NT), jnp.float32),
        ],
        scratch_shapes=[
            pltpu.VMEM((1, NT), jnp.float32),
            pltpu.VMEM((1, NT), jnp.float32),
        ],
        interpret=interpret,
    )(q2, lt)


def _sc_scatter_x(xpad, slotof):
    mesh = plsc.VectorSubcoreMesh(core_axis_name="c", subcore_axis_name="s")

    @functools.partial(
        pl.kernel,
        out_type=jax.ShapeDtypeStruct((B, DPAD), jnp.float32),
        mesh=mesh,
        scratch_types=[
            pltpu.VMEM((BPW,), jnp.int32),
            pltpu.VMEM((BPW, DPAD), jnp.float32),
            pltpu.SemaphoreType.DMA,
        ],
    )
    def sk(xpad_hbm, slotof_hbm, xs_hbm, ids_v, rows_v, sem):
        wid = lax.axis_index("s") * N_SC_CORES + lax.axis_index("c")
        base = wid * BPW
        pltpu.sync_copy(slotof_hbm.at[pl.ds(base, BPW)], ids_v)
        pltpu.sync_copy(xpad_hbm.at[pl.ds(base, BPW)], rows_v)
        pltpu.async_copy(rows_v, xs_hbm.at[ids_v], sem).wait()

    return sk(xpad, slotof)


def _table_body(tab_ref, sidx_ref, tablen_ref, ul_ref):
    tab = tab_ref[...]
    tn = tab / jnp.maximum(
        jnp.sqrt(jnp.sum(tab * tab, axis=1, keepdims=True)), 1e-12)
    tablen_ref[...] = jnp.concatenate(
        [tn, jnp.zeros((K, DPAD - D), jnp.float32)], axis=1)
    si = sidx_ref[...]
    onehot = (si.reshape(SPAD, 1)
              == lax.broadcasted_iota(jnp.int32, (SPAD, K), 1))
    se = lax.dot_general(onehot.astype(jnp.float32), tn,
                         (((1,), (0,)), ((), ())),
                         preferred_element_type=jnp.float32,
                         precision=lax.Precision.HIGHEST)
    sim = lax.dot_general(se, se, (((1,), (1,)), ((), ())),
                          preferred_element_type=jnp.float32,
                          precision=lax.Precision.HIGHEST)
    valid = si.reshape(1, SPAD) >= 0
    eye = (lax.broadcasted_iota(jnp.int32, (SPAD, SPAD), 0)
           == lax.broadcasted_iota(jnp.int32, (SPAD, SPAD), 1))
    keep = valid & jnp.logical_not(eye)
    simm = jnp.where(keep, sim, -jnp.float32(jnp.inf))
    ex = jnp.exp(simm / TEMP)
    sum_exp = jnp.sum(ex, axis=1, keepdims=True)
    labels = jnp.where(si >= 0, si // PER, -1)
    pos = labels.reshape(SPAD, 1) == labels.reshape(1, SPAD)
    pos_sum = jnp.sum(jnp.where(pos, ex, 0.0), axis=1, keepdims=True)
    validc = si.reshape(SPAD, 1) >= 0
    ratio = jnp.where(validc, pos_sum / jnp.maximum(sum_exp, 1e-30), 1.0)
    ul = -jnp.sum(jnp.log(ratio)) / SAMPLED
    ul_ref[...] = jnp.full((1, 1), ul, dtype=jnp.float32)


def _tc_table(table, sidx, interpret=False):
    return pl.pallas_call(
        _table_body,
        grid=(1,),
        in_specs=[
            pl.BlockSpec((K, D), lambda i: (0, 0)),
            pl.BlockSpec((1, SPAD), lambda i: (0, 0)),
        ],
        out_specs=[
            pl.BlockSpec((K, DPAD), lambda i: (0, 0)),
            pl.BlockSpec((1, 1), lambda i: (0, 0)),
        ],
        out_shape=[
            jax.ShapeDtypeStruct((K, DPAD), jnp.float32),
            jax.ShapeDtypeStruct((1, 1), jnp.float32),
        ],
        interpret=interpret,
    )(table, sidx)


def _bucket_body(xs_ref, starts_ref, tab_ref, e2_ref, enc_ref):
    step = pl.program_id(0)

    xr = xs_ref[:, :D]
    norm = jnp.sqrt(jnp.sum(xr * xr, axis=1, keepdims=True))
    xn = xr / jnp.maximum(norm, 1e-12)
    c = jnp.sum(xn * xn, axis=1, keepdims=True)

    fslot = (lax.broadcasted_iota(jnp.int32, (TBS, 1), 0)
             + step * TBS).astype(jnp.float32)
    qs = (jnp.sum((fslot >= starts_ref[...]).astype(jnp.int32),
                  axis=1, keepdims=True) - 1)
    t_lo = jnp.min(qs)
    t_hi = jnp.max(qs)

    def tb(t, best):
        st = tab_ref[pl.ds(t * PER, PER), :]
        sc = lax.dot_general(xn, st, (((1,), (1,)), ((), ())),
                             preferred_element_type=jnp.float32)
        e2r = e2_ref[pl.ds(t * 8, 8), :][0:1, :]
        d2b = (c + e2r) - 2.0 * sc
        mb = jnp.min(d2b, axis=1, keepdims=True)
        lane = lax.broadcasted_iota(jnp.int32, (TBS, PER), 1)
        ib = jnp.min(jnp.where(d2b == mb, lane, PER), axis=1, keepdims=True)
        return jnp.where(qs == t, t * PER + ib, best)

    best = lax.fori_loop(t_lo, t_hi + 1, tb, jnp.zeros((TBS, 1), jnp.int32))
    enc_ref[...] = best


def _tc_bucket(xs, starts, table, e2rep, interpret=False):
    return pl.pallas_call(
        _bucket_body,
        grid=(B // TBS,),
        in_specs=[
            pl.BlockSpec((TBS, DPAD), lambda i: (i, 0)),
            pl.BlockSpec((1, NT), lambda i: (0, 0)),
            pl.BlockSpec((K, D), lambda i: (0, 0)),
            pl.BlockSpec((NUM_PTM * 8, PER), lambda i: (0, 0)),
        ],
        out_specs=pl.BlockSpec((TBS, 1), lambda i: (i, 0)),
        out_shape=jax.ShapeDtypeStruct((B, 1), jnp.int32),
        interpret=interpret,
    )(xs, starts, table, e2rep)


def _sc_gather_out(tablen, encs, slotof):
    mesh = plsc.VectorSubcoreMesh(core_axis_name="c", subcore_axis_name="s")

    @functools.partial(
        pl.kernel,
        out_type=[jax.ShapeDtypeStruct((B,), jnp.int32),
                  jax.ShapeDtypeStruct((B, DPAD), jnp.float32)],
        mesh=mesh,
        scratch_types=[
            pltpu.VMEM((BPW,), jnp.int32),
            pltpu.VMEM((BPW,), jnp.int32),
            pltpu.VMEM((BPW, DPAD), jnp.float32),
            pltpu.SemaphoreType.DMA,
            pltpu.SemaphoreType.DMA,
        ],
    )
    def gk(tablen_hbm, encs_hbm, slotof_hbm, enc_hbm, quant_hbm,
           ids_v, enc_v, rows_v, sem1, sem2):
        wid = lax.axis_index("s") * N_SC_CORES + lax.axis_index("c")
        base = wid * BPW
        pltpu.sync_copy(slotof_hbm.at[pl.ds(base, BPW)], ids_v)
        pltpu.async_copy(encs_hbm.at[ids_v], enc_v, sem1).wait()
        pltpu.sync_copy(enc_v, enc_hbm.at[pl.ds(base, BPW)])
        pltpu.async_copy(tablen_hbm.at[enc_v], rows_v, sem2).wait()
        pltpu.sync_copy(rows_v, quant_hbm.at[pl.ds(base, BPW)])

    return gk(tablen, encs, slotof)


def _loss_body(x_ref, qp_ref, lsum_ref):
    step = pl.program_id(0)
    xr = x_ref[...]
    norm = jnp.sqrt(jnp.sum(xr * xr, axis=1, keepdims=True))
    xn = xr / jnp.maximum(norm, 1e-12)
    diff = qp_ref[:, :D] - xn

    @pl.when(step == 0)
    def _():
        lsum_ref[...] = jnp.zeros_like(lsum_ref)

    lsum_ref[...] += jnp.sum(diff * diff).reshape(1, 1)


def _tc_loss(x, qp, interpret=False):
    return pl.pallas_call(
        _loss_body,
        grid=(B // TBL,),
        in_specs=[
            pl.BlockSpec((TBL, D), lambda i: (i, 0)),
            pl.BlockSpec((TBL, DPAD), lambda i: (i, 0)),
        ],
        out_specs=pl.BlockSpec((1, 1), lambda i: (0, 0)),
        out_shape=jax.ShapeDtypeStruct((1, 1), jnp.float32),
        interpret=interpret,
    )(x, qp)


def _sampled_indices():
    perm = jax.random.permutation(jax.random.key(42), PER)[:int(0.1 * PER)]
    all_idx = jnp.arange(K).reshape(NUM_PTM, PER)
    si = all_idx[:, perm].reshape(-1).astype(jnp.int32)
    return jnp.concatenate(
        [si, jnp.full((SPAD - SAMPLED,), -1, jnp.int32)]).reshape(1, SPAD)


def kernel(x, Q, embeddings):
    q2 = Q.reshape(B, 1)
    xpad = jnp.concatenate([x, jnp.zeros((B, DPAD - D), x.dtype)], axis=1)
    e2rep = jnp.repeat(jnp.sum(embeddings ** 2, axis=1).reshape(NUM_PTM, PER),
                       8, axis=0)
    sidx = _sampled_indices()
    lt = jnp.tril(jnp.ones((TBP, TBP), jnp.float32))

    slotof2, starts = _tc_perm(q2, lt)
    slotof = slotof2.reshape(B)
    xs = _sc_scatter_x(xpad, slotof)
    tablen, ul = _tc_table(embeddings, sidx)
    encs2 = _tc_bucket(xs, starts, embeddings, e2rep)
    enc, qp = _sc_gather_out(tablen, encs2.reshape(B), slotof)
    lsum = _tc_loss(x, qp)

    quantized = qp[:, :D]
    loss = lsum[0, 0] * ((1.0 + COMMIT) / (B * D))
    return (quantized, loss, ul[0, 0], enc)


# speedup vs baseline: 1.0561x; 1.0173x over previous
import functools

import jax
import jax.numpy as jnp
from jax import lax
from jax.experimental import pallas as pl
from jax.experimental.pallas import tpu as pltpu
from jax.experimental.pallas import tpu_sc as plsc

B = 16384
D = 64
NUM_PTM = 26
PER = 128
K = NUM_PTM * PER
COMMIT = 0.25
TEMP = 0.07
SAMPLED = int(0.1 * PER) * NUM_PTM
SPAD = 384

DPAD = 128
TBP = 512
TBS = 1024
TBL = 4096
NT = 32
N_SC_CORES = 2
N_SC_SUBCORES = 16
NW = N_SC_CORES * N_SC_SUBCORES
BPW = B // NW


def _perm_body(q_ref, lt_ref, slot_ref, starts_ref, hist_ref, carry_ref):
    i = pl.program_id(0)
    nbp = B // TBP
    t_iota = lax.broadcasted_iota(jnp.int32, (1, NT), 1)
    oneh = (q_ref[...] == t_iota).astype(jnp.float32)

    @pl.when(i == 0)
    def _():
        hist_ref[...] = jnp.zeros_like(hist_ref)

    @pl.when(i < nbp)
    def _():
        hist_ref[...] += jnp.sum(oneh, axis=0, keepdims=True)

    @pl.when(i == nbp)
    def _():
        h = hist_ref[...]
        strict_lt = (lax.broadcasted_iota(jnp.int32, (NT, NT), 0)
                     < lax.broadcasted_iota(jnp.int32, (NT, NT), 1))
        st = lax.dot_general(h, strict_lt.astype(jnp.float32),
                             (((1,), (0,)), ((), ())),
                             preferred_element_type=jnp.float32,
                             precision=lax.Precision.HIGHEST)
        starts_ref[...] = st
        carry_ref[...] = st

    @pl.when(i >= nbp)
    def _():
        pref = lax.dot_general(lt_ref[...], oneh, (((1,), (0,)), ((), ())),
                               preferred_element_type=jnp.float32,
                               precision=lax.Precision.HIGHEST)
        base = carry_ref[...]
        pos = jnp.sum(oneh * (base + pref), axis=1, keepdims=True) - 1.0
        slot_ref[...] = pos.astype(jnp.int32)
        carry_ref[...] = base + jnp.sum(oneh, axis=0, keepdims=True)


def _tc_perm(q2, lt, interpret=False):
    nbp = B // TBP
    return pl.pallas_call(
        _perm_body,
        grid=(2 * nbp,),
        in_specs=[
            pl.BlockSpec((TBP, 1), lambda i: (i % (B // TBP), 0)),
            pl.BlockSpec((TBP, TBP), lambda i: (0, 0)),
        ],
        out_specs=[
            pl.BlockSpec((TBP, 1), lambda i: (i % (B // TBP), 0)),
            pl.BlockSpec((1, NT), lambda i: (0, 0)),
        ],
        out_shape=[
            jax.ShapeDtypeStruct((B, 1), jnp.int32),
            jax.ShapeDtypeStruct((1, NT), jnp.float32),
        ],
        scratch_shapes=[
            pltpu.VMEM((1, NT), jnp.float32),
            pltpu.VMEM((1, NT), jnp.float32),
        ],
        interpret=interpret,
    )(q2, lt)


def _sc_scatter_x(xpad, slotof):
    mesh = plsc.VectorSubcoreMesh(core_axis_name="c", subcore_axis_name="s")

    @functools.partial(
        pl.kernel,
        out_type=jax.ShapeDtypeStruct((B, DPAD), jnp.float32),
        mesh=mesh,
        scratch_types=[
            pltpu.VMEM((BPW,), jnp.int32),
            pltpu.VMEM((BPW, DPAD), jnp.float32),
            pltpu.SemaphoreType.DMA,
        ],
    )
    def sk(xpad_hbm, slotof_hbm, xs_hbm, ids_v, rows_v, sem):
        wid = lax.axis_index("s") * N_SC_CORES + lax.axis_index("c")
        base = wid * BPW
        pltpu.sync_copy(slotof_hbm.at[pl.ds(base, BPW)], ids_v)
        pltpu.sync_copy(xpad_hbm.at[pl.ds(base, BPW)], rows_v)
        pltpu.async_copy(rows_v, xs_hbm.at[ids_v], sem).wait()

    return sk(xpad, slotof)


def _table_body(tab_ref, sidx_ref, tablen_ref, ul_ref):
    tab = tab_ref[...]
    tn = tab / jnp.maximum(
        jnp.sqrt(jnp.sum(tab * tab, axis=1, keepdims=True)), 1e-12)
    tablen_ref[...] = jnp.concatenate(
        [tn, jnp.zeros((K, DPAD - D), jnp.float32)], axis=1)
    si = sidx_ref[...]
    onehot = (si.reshape(SPAD, 1)
              == lax.broadcasted_iota(jnp.int32, (SPAD, K), 1))
    se = lax.dot_general(onehot.astype(jnp.float32), tn,
                         (((1,), (0,)), ((), ())),
                         preferred_element_type=jnp.float32,
                         precision=lax.Precision.HIGHEST)
    sim = lax.dot_general(se, se, (((1,), (1,)), ((), ())),
                          preferred_element_type=jnp.float32,
                          precision=lax.Precision.HIGHEST)
    valid = si.reshape(1, SPAD) >= 0
    eye = (lax.broadcasted_iota(jnp.int32, (SPAD, SPAD), 0)
           == lax.broadcasted_iota(jnp.int32, (SPAD, SPAD), 1))
    keep = valid & jnp.logical_not(eye)
    simm = jnp.where(keep, sim, -jnp.float32(jnp.inf))
    ex = jnp.exp(simm / TEMP)
    sum_exp = jnp.sum(ex, axis=1, keepdims=True)
    labels = jnp.where(si >= 0, si // PER, -1)
    pos = labels.reshape(SPAD, 1) == labels.reshape(1, SPAD)
    pos_sum = jnp.sum(jnp.where(pos, ex, 0.0), axis=1, keepdims=True)
    validc = si.reshape(SPAD, 1) >= 0
    ratio = jnp.where(validc, pos_sum / jnp.maximum(sum_exp, 1e-30), 1.0)
    ul = -jnp.sum(jnp.log(ratio)) / SAMPLED
    ul_ref[...] = jnp.full((1, 1), ul, dtype=jnp.float32)


def _tc_table(table, sidx, interpret=False):
    return pl.pallas_call(
        _table_body,
        grid=(1,),
        in_specs=[
            pl.BlockSpec((K, D), lambda i: (0, 0)),
            pl.BlockSpec((1, SPAD), lambda i: (0, 0)),
        ],
        out_specs=[
            pl.BlockSpec((K, DPAD), lambda i: (0, 0)),
            pl.BlockSpec((1, 1), lambda i: (0, 0)),
        ],
        out_shape=[
            jax.ShapeDtypeStruct((K, DPAD), jnp.float32),
            jax.ShapeDtypeStruct((1, 1), jnp.float32),
        ],
        interpret=interpret,
    )(table, sidx)


def _bucket_body(xs_ref, starts_ref, tab_ref, e2_ref, enc_ref, lsum_ref):
    step = pl.program_id(0)

    xr = xs_ref[:, :D]
    norm = jnp.sqrt(jnp.sum(xr * xr, axis=1, keepdims=True))
    xn = xr / jnp.maximum(norm, 1e-12)
    c = jnp.sum(xn * xn, axis=1, keepdims=True)

    fslot = (lax.broadcasted_iota(jnp.int32, (TBS, 1), 0)
             + step * TBS).astype(jnp.float32)
    qs = (jnp.sum((fslot >= starts_ref[...]).astype(jnp.int32),
                  axis=1, keepdims=True) - 1)
    t_lo = jnp.min(qs)
    t_hi = jnp.max(qs)

    def tb(t, carry):
        best, bm = carry
        st = tab_ref[pl.ds(t * PER, PER), :]
        sc = lax.dot_general(xn, st, (((1,), (1,)), ((), ())),
                             preferred_element_type=jnp.float32)
        e2r = e2_ref[pl.ds(t * 8, 8), :][0:1, :]
        d2b = (c + e2r) - 2.0 * sc
        mb = jnp.min(d2b, axis=1, keepdims=True)
        lane = lax.broadcasted_iota(jnp.int32, (TBS, PER), 1)
        ib = jnp.min(jnp.where(d2b == mb, lane, PER), axis=1, keepdims=True)
        upd = qs == t
        return (jnp.where(upd, t * PER + ib, best), jnp.where(upd, mb, bm))

    best, bm = lax.fori_loop(
        t_lo, t_hi + 1, tb,
        (jnp.zeros((TBS, 1), jnp.int32), jnp.zeros((TBS, 1), jnp.float32)))
    enc_ref[...] = best

    @pl.when(step == 0)
    def _():
        lsum_ref[...] = jnp.zeros_like(lsum_ref)

    lsum_ref[...] += jnp.sum(bm).reshape(1, 1)


def _tc_bucket(xs, starts, table, e2rep, interpret=False):
    return pl.pallas_call(
        _bucket_body,
        grid=(B // TBS,),
        in_specs=[
            pl.BlockSpec((TBS, DPAD), lambda i: (i, 0)),
            pl.BlockSpec((1, NT), lambda i: (0, 0)),
            pl.BlockSpec((K, D), lambda i: (0, 0)),
            pl.BlockSpec((NUM_PTM * 8, PER), lambda i: (0, 0)),
        ],
        out_specs=[pl.BlockSpec((TBS, 1), lambda i: (i, 0)),
                   pl.BlockSpec((1, 1), lambda i: (0, 0))],
        out_shape=[jax.ShapeDtypeStruct((B, 1), jnp.int32),
                   jax.ShapeDtypeStruct((1, 1), jnp.float32)],
        interpret=interpret,
    )(xs, starts, table, e2rep)


def _sc_gather_out(tablen, encs, slotof):
    mesh = plsc.VectorSubcoreMesh(core_axis_name="c", subcore_axis_name="s")

    @functools.partial(
        pl.kernel,
        out_type=[jax.ShapeDtypeStruct((B,), jnp.int32),
                  jax.ShapeDtypeStruct((B, DPAD), jnp.float32)],
        mesh=mesh,
        scratch_types=[
            pltpu.VMEM((BPW,), jnp.int32),
            pltpu.VMEM((BPW,), jnp.int32),
            pltpu.VMEM((BPW, DPAD), jnp.float32),
            pltpu.SemaphoreType.DMA,
            pltpu.SemaphoreType.DMA,
        ],
    )
    def gk(tablen_hbm, encs_hbm, slotof_hbm, enc_hbm, quant_hbm,
           ids_v, enc_v, rows_v, sem1, sem2):
        wid = lax.axis_index("s") * N_SC_CORES + lax.axis_index("c")
        base = wid * BPW
        pltpu.sync_copy(slotof_hbm.at[pl.ds(base, BPW)], ids_v)
        pltpu.async_copy(encs_hbm.at[ids_v], enc_v, sem1).wait()
        pltpu.sync_copy(enc_v, enc_hbm.at[pl.ds(base, BPW)])
        pltpu.async_copy(tablen_hbm.at[enc_v], rows_v, sem2).wait()
        pltpu.sync_copy(rows_v, quant_hbm.at[pl.ds(base, BPW)])

    return gk(tablen, encs, slotof)


def _loss_body(x_ref, qp_ref, lsum_ref):
    step = pl.program_id(0)
    xr = x_ref[...]
    norm = jnp.sqrt(jnp.sum(xr * xr, axis=1, keepdims=True))
    xn = xr / jnp.maximum(norm, 1e-12)
    diff = qp_ref[:, :D] - xn

    @pl.when(step == 0)
    def _():
        lsum_ref[...] = jnp.zeros_like(lsum_ref)

    lsum_ref[...] += jnp.sum(diff * diff).reshape(1, 1)


def _tc_loss(x, qp, interpret=False):
    return pl.pallas_call(
        _loss_body,
        grid=(B // TBL,),
        in_specs=[
            pl.BlockSpec((TBL, D), lambda i: (i, 0)),
            pl.BlockSpec((TBL, DPAD), lambda i: (i, 0)),
        ],
        out_specs=pl.BlockSpec((1, 1), lambda i: (0, 0)),
        out_shape=jax.ShapeDtypeStruct((1, 1), jnp.float32),
        interpret=interpret,
    )(x, qp)


def _sampled_indices():
    perm = jax.random.permutation(jax.random.key(42), PER)[:int(0.1 * PER)]
    all_idx = jnp.arange(K).reshape(NUM_PTM, PER)
    si = all_idx[:, perm].reshape(-1).astype(jnp.int32)
    return jnp.concatenate(
        [si, jnp.full((SPAD - SAMPLED,), -1, jnp.int32)]).reshape(1, SPAD)


def kernel(x, Q, embeddings):
    q2 = Q.reshape(B, 1)
    xpad = jnp.concatenate([x, jnp.zeros((B, DPAD - D), x.dtype)], axis=1)
    e2rep = jnp.repeat(jnp.sum(embeddings ** 2, axis=1).reshape(NUM_PTM, PER),
                       8, axis=0)
    sidx = _sampled_indices()
    lt = jnp.tril(jnp.ones((TBP, TBP), jnp.float32))

    slotof2, starts = _tc_perm(q2, lt)
    slotof = slotof2.reshape(B)
    xs = _sc_scatter_x(xpad, slotof)
    tablen, ul = _tc_table(embeddings, sidx)
    encs2, lsum = _tc_bucket(xs, starts, embeddings, e2rep)
    enc, qp = _sc_gather_out(tablen, encs2.reshape(B), slotof)

    quantized = qp[:, :D]
    loss = lsum[0, 0] * ((1.0 + COMMIT) / (B * D))
    return (quantized, loss, ul[0, 0], enc)


# speedup vs baseline: 1.3353x; 1.2644x over previous
import functools

import jax
import jax.numpy as jnp
from jax import lax
from jax.experimental import pallas as pl
from jax.experimental.pallas import tpu as pltpu
from jax.experimental.pallas import tpu_sc as plsc

B = 16384
D = 64
NUM_PTM = 26
PER = 128
K = NUM_PTM * PER
COMMIT = 0.25
TEMP = 0.07
SAMPLED = int(0.1 * PER) * NUM_PTM
SPAD = 384

TB = 2048
DPAD = 128
N_SC_CORES = 2
N_SC_SUBCORES = 16
NW = N_SC_CORES * N_SC_SUBCORES
BPW = B // NW


def _tc_body(x_ref, q_ref, tab_ref, e2_ref,
             idx_ref, lsum_ref):
    step = pl.program_id(0)

    xr = x_ref[...]
    norm = jnp.sqrt(jnp.sum(xr * xr, axis=1, keepdims=True))
    xn = xr / jnp.maximum(norm, 1e-12)
    c = jnp.sum(xn * xn, axis=1, keepdims=True)

    s = lax.dot_general(xn, tab_ref[...], (((1,), (1,)), ((), ())),
                        preferred_element_type=jnp.float32)
    d2 = (c + e2_ref[...]) - 2.0 * s

    coltype = lax.broadcasted_iota(jnp.int32, (1, K), 1) // PER
    mask = coltype == q_ref[...]
    inf = jnp.float32(jnp.inf)
    d2m = jnp.where(mask, d2, inf)
    m = jnp.min(d2m, axis=1, keepdims=True)
    lane = lax.broadcasted_iota(jnp.int32, (TB, K), 1)
    idx = jnp.min(jnp.where(d2m == m, lane, K), axis=1)
    idx_ref[...] = idx[:, None]

    @pl.when(step == 0)
    def _():
        lsum_ref[...] = jnp.zeros_like(lsum_ref)

    lsum_ref[...] += jnp.sum(m).reshape(1, 1)


def _tc_call(x, q2, table, e2, interpret=False):
    grid = B // TB
    return pl.pallas_call(
        _tc_body,
        grid=(grid,),
        in_specs=[
            pl.BlockSpec((TB, D), lambda i: (i, 0)),
            pl.BlockSpec((TB, 1), lambda i: (i, 0)),
            pl.BlockSpec((K, D), lambda i: (0, 0)),
            pl.BlockSpec((1, K), lambda i: (0, 0)),
        ],
        out_specs=[
            pl.BlockSpec((TB, 1), lambda i: (i, 0)),
            pl.BlockSpec((1, 1), lambda i: (0, 0)),
        ],
        out_shape=[
            jax.ShapeDtypeStruct((B, 1), jnp.int32),
            jax.ShapeDtypeStruct((1, 1), jnp.float32),
        ],
        interpret=interpret,
    )(x, q2, table, e2)


def _table_body(tab_ref, sidx_ref, tablen_ref, ul_ref):
    tab = tab_ref[...]
    tn = tab / jnp.maximum(
        jnp.sqrt(jnp.sum(tab * tab, axis=1, keepdims=True)), 1e-12)
    tablen_ref[...] = jnp.concatenate(
        [tn, jnp.zeros((K, DPAD - D), jnp.float32)], axis=1)
    si = sidx_ref[...]
    onehot = (si.reshape(SPAD, 1)
              == lax.broadcasted_iota(jnp.int32, (SPAD, K), 1))
    se = lax.dot_general(onehot.astype(jnp.float32), tn,
                         (((1,), (0,)), ((), ())),
                         preferred_element_type=jnp.float32,
                         precision=lax.Precision.HIGHEST)
    sim = lax.dot_general(se, se, (((1,), (1,)), ((), ())),
                          preferred_element_type=jnp.float32,
                          precision=lax.Precision.HIGHEST)
    valid = si.reshape(1, SPAD) >= 0
    eye = (lax.broadcasted_iota(jnp.int32, (SPAD, SPAD), 0)
           == lax.broadcasted_iota(jnp.int32, (SPAD, SPAD), 1))
    keep = valid & jnp.logical_not(eye)
    simm = jnp.where(keep, sim, -jnp.float32(jnp.inf))
    ex = jnp.exp(simm / TEMP)
    sum_exp = jnp.sum(ex, axis=1, keepdims=True)
    labels = jnp.where(si >= 0, si // PER, -1)
    pos = labels.reshape(SPAD, 1) == labels.reshape(1, SPAD)
    pos_sum = jnp.sum(jnp.where(pos, ex, 0.0), axis=1, keepdims=True)
    validc = si.reshape(SPAD, 1) >= 0
    ratio = jnp.where(validc, pos_sum / jnp.maximum(sum_exp, 1e-30), 1.0)
    ul = -jnp.sum(jnp.log(ratio)) / SAMPLED
    ul_ref[...] = jnp.full((1, 1), ul, dtype=jnp.float32)


def _tc_table(table, sidx, interpret=False):
    return pl.pallas_call(
        _table_body,
        grid=(1,),
        in_specs=[
            pl.BlockSpec((K, D), lambda i: (0, 0)),
            pl.BlockSpec((1, SPAD), lambda i: (0, 0)),
        ],
        out_specs=[
            pl.BlockSpec((K, DPAD), lambda i: (0, 0)),
            pl.BlockSpec((1, 1), lambda i: (0, 0)),
        ],
        out_shape=[
            jax.ShapeDtypeStruct((K, DPAD), jnp.float32),
            jax.ShapeDtypeStruct((1, 1), jnp.float32),
        ],
        interpret=interpret,
    )(table, sidx)


def _sc_gather(tablen, idx):
    mesh = plsc.VectorSubcoreMesh(core_axis_name="c", subcore_axis_name="s")

    @functools.partial(
        pl.kernel,
        out_type=jax.ShapeDtypeStruct((B, DPAD), jnp.float32),
        mesh=mesh,
        scratch_types=[
            pltpu.VMEM((BPW,), jnp.int32),
            pltpu.VMEM((BPW, DPAD), jnp.float32),
            pltpu.SemaphoreType.DMA,
        ],
    )
    def gk(table_hbm, idx_hbm, out_hbm, idx_v, rows_v, sem):
        wid = lax.axis_index("s") * N_SC_CORES + lax.axis_index("c")
        base = wid * BPW
        pltpu.sync_copy(idx_hbm.at[pl.ds(base, BPW)], idx_v)
        pltpu.async_copy(table_hbm.at[idx_v], rows_v, sem).wait()
        pltpu.sync_copy(rows_v, out_hbm.at[pl.ds(base, BPW)])

    return gk(tablen, idx)


def _sampled_indices():
    perm = jax.random.permutation(jax.random.key(42), PER)[:int(0.1 * PER)]
    all_idx = jnp.arange(K).reshape(NUM_PTM, PER)
    si = all_idx[:, perm].reshape(-1).astype(jnp.int32)
    return jnp.concatenate(
        [si, jnp.full((SPAD - SAMPLED,), -1, jnp.int32)]).reshape(1, SPAD)


def kernel(x, Q, embeddings):
    e2 = jnp.sum(embeddings ** 2, axis=1)[None, :]
    q2 = Q.reshape(B, 1)
    sidx = _sampled_indices()
    idx2, lsum = _tc_call(x, q2, embeddings, e2)
    tablen, ul = _tc_table(embeddings, sidx)
    idx = idx2.reshape(B)
    quantized = _sc_gather(tablen, idx)[:, :D]
    loss = lsum[0, 0] * ((1.0 + COMMIT) / (B * D))
    return (quantized, loss, ul[0, 0], idx)


# speedup vs baseline: 1.3466x; 1.0085x over previous
import functools

import jax
import jax.numpy as jnp
from jax import lax
from jax.experimental import pallas as pl
from jax.experimental.pallas import tpu as pltpu
from jax.experimental.pallas import tpu_sc as plsc

B = 16384
D = 64
NUM_PTM = 26
PER = 128
K = NUM_PTM * PER
COMMIT = 0.25
TEMP = 0.07
SAMPLED = int(0.1 * PER) * NUM_PTM
SPAD = 384

TB = 2048
DPAD = 128
N_SC_CORES = 2
N_SC_SUBCORES = 16
NW = N_SC_CORES * N_SC_SUBCORES
BPW = B // NW


def _tc_body(x_ref, q_ref, tab_ref, e2_ref, sidx_ref,
             idx_ref, tablen_ref, lsum_ref, ul_ref):
    step = pl.program_id(0)

    xr = x_ref[...]
    norm = jnp.sqrt(jnp.sum(xr * xr, axis=1, keepdims=True))
    xn = xr / jnp.maximum(norm, 1e-12)
    c = jnp.sum(xn * xn, axis=1, keepdims=True)

    s = lax.dot_general(xn, tab_ref[...], (((1,), (1,)), ((), ())),
                        preferred_element_type=jnp.float32)
    d2 = (c + e2_ref[...]) - 2.0 * s

    coltype = lax.broadcasted_iota(jnp.int32, (1, K), 1) // PER
    mask = coltype == q_ref[...]
    inf = jnp.float32(jnp.inf)
    d2m = jnp.where(mask, d2, inf)
    m = jnp.min(d2m, axis=1, keepdims=True)
    lane = lax.broadcasted_iota(jnp.int32, (TB, K), 1)
    idx = jnp.min(jnp.where(d2m == m, lane, K), axis=1)
    idx_ref[...] = idx[:, None]

    @pl.when(step == 0)
    def _():
        lsum_ref[...] = jnp.zeros_like(lsum_ref)
        tab = tab_ref[...]
        tn = tab / jnp.maximum(
            jnp.sqrt(jnp.sum(tab * tab, axis=1, keepdims=True)), 1e-12)
        tablen_ref[...] = jnp.concatenate(
            [tn, jnp.zeros((K, DPAD - D), jnp.float32)], axis=1)
        si = sidx_ref[...]
        onehot = (si.reshape(SPAD, 1)
                  == lax.broadcasted_iota(jnp.int32, (SPAD, K), 1))
        se = lax.dot_general(onehot.astype(jnp.float32), tn,
                             (((1,), (0,)), ((), ())),
                             preferred_element_type=jnp.float32,
                             precision=lax.Precision.HIGHEST)
        sim = lax.dot_general(se, se, (((1,), (1,)), ((), ())),
                              preferred_element_type=jnp.float32,
                              precision=lax.Precision.HIGHEST)
        valid = si.reshape(1, SPAD) >= 0
        eye = (lax.broadcasted_iota(jnp.int32, (SPAD, SPAD), 0)
               == lax.broadcasted_iota(jnp.int32, (SPAD, SPAD), 1))
        keep = valid & jnp.logical_not(eye)
        simm = jnp.where(keep, sim, -jnp.float32(jnp.inf))
        ex = jnp.exp(simm / TEMP)
        sum_exp = jnp.sum(ex, axis=1, keepdims=True)
        labels = jnp.where(si >= 0, si // PER, -1)
        pos = labels.reshape(SPAD, 1) == labels.reshape(1, SPAD)
        pos_sum = jnp.sum(jnp.where(pos, ex, 0.0), axis=1, keepdims=True)
        validc = si.reshape(SPAD, 1) >= 0
        ratio = jnp.where(validc, pos_sum / jnp.maximum(sum_exp, 1e-30), 1.0)
        ul = -jnp.sum(jnp.log(ratio)) / SAMPLED
        ul_ref[...] = jnp.full((1, 1), ul, dtype=jnp.float32)

    lsum_ref[...] += jnp.sum(m).reshape(1, 1)


def _tc_call(x, q2, table, e2, sidx, interpret=False):
    grid = B // TB
    return pl.pallas_call(
        _tc_body,
        grid=(grid,),
        in_specs=[
            pl.BlockSpec((TB, D), lambda i: (i, 0)),
            pl.BlockSpec((TB, 1), lambda i: (i, 0)),
            pl.BlockSpec((K, D), lambda i: (0, 0)),
            pl.BlockSpec((1, K), lambda i: (0, 0)),
            pl.BlockSpec((1, SPAD), lambda i: (0, 0)),
        ],
        out_specs=[
            pl.BlockSpec((TB, 1), lambda i: (i, 0)),
            pl.BlockSpec((K, DPAD), lambda i: (0, 0)),
            pl.BlockSpec((1, 1), lambda i: (0, 0)),
            pl.BlockSpec((1, 1), lambda i: (0, 0)),
        ],
        out_shape=[
            jax.ShapeDtypeStruct((B, 1), jnp.int32),
            jax.ShapeDtypeStruct((K, DPAD), jnp.float32),
            jax.ShapeDtypeStruct((1, 1), jnp.float32),
            jax.ShapeDtypeStruct((1, 1), jnp.float32),
        ],
        interpret=interpret,
    )(x, q2, table, e2, sidx)


def _sc_gather(tablen, idx):
    mesh = plsc.VectorSubcoreMesh(core_axis_name="c", subcore_axis_name="s")

    @functools.partial(
        pl.kernel,
        out_type=jax.ShapeDtypeStruct((B, DPAD), jnp.float32),
        mesh=mesh,
        scratch_types=[
            pltpu.VMEM((BPW,), jnp.int32),
            pltpu.VMEM((BPW, DPAD), jnp.float32),
            pltpu.SemaphoreType.DMA,
        ],
    )
    def gk(table_hbm, idx_hbm, out_hbm, idx_v, rows_v, sem):
        wid = lax.axis_index("s") * N_SC_CORES + lax.axis_index("c")
        base = wid * BPW
        pltpu.sync_copy(idx_hbm.at[pl.ds(base, BPW)], idx_v)
        pltpu.async_copy(table_hbm.at[idx_v], rows_v, sem).wait()
        pltpu.sync_copy(rows_v, out_hbm.at[pl.ds(base, BPW)])

    return gk(tablen, idx)


def _sampled_indices():
    perm = jax.random.permutation(jax.random.key(42), PER)[:int(0.1 * PER)]
    all_idx = jnp.arange(K).reshape(NUM_PTM, PER)
    si = all_idx[:, perm].reshape(-1).astype(jnp.int32)
    return jnp.concatenate(
        [si, jnp.full((SPAD - SAMPLED,), -1, jnp.int32)]).reshape(1, SPAD)


def kernel(x, Q, embeddings):
    e2 = jnp.sum(embeddings ** 2, axis=1)[None, :]
    q2 = Q.reshape(B, 1)
    sidx = _sampled_indices()
    idx2, tablen, lsum, ul = _tc_call(x, q2, embeddings, e2, sidx)
    idx = idx2.reshape(B)
    quantized = _sc_gather(tablen, idx)[:, :D]
    loss = lsum[0, 0] * ((1.0 + COMMIT) / (B * D))
    return (quantized, loss, ul[0, 0], idx)
